# baseline (speedup 1.0000x reference)
import functools

import jax
import jax.numpy as jnp
from jax import lax
from jax.experimental import pallas as pl
from jax.experimental.pallas import tpu as pltpu
from jax.experimental.pallas import tpu_sc as plsc

B = 64
D = 1024
HID = 1024
OUT = 1024
E = 64
K = 2

_LANE = 16
_NSUB = 16
_ROWS = B // _NSUB


def _gate_body(x_ref, wg_ref, bg_ref, probs_ref, g_ref, act_ref):
    h = jnp.dot(x_ref[...], wg_ref[...],
                preferred_element_type=jnp.float32) + bg_ref[...]
    m1 = jnp.max(h, axis=1, keepdims=True)
    e_all = jnp.exp(h - m1)
    probs_ref[...] = e_all / jnp.sum(e_all, axis=1, keepdims=True)
    is_max = h == m1
    cnt = jnp.sum(is_max.astype(jnp.float32), axis=1, keepdims=True)
    m2 = jnp.max(jnp.where(is_max, -jnp.inf, h), axis=1, keepdims=True)
    kth = jnp.where(cnt >= 2.0, m1, m2)
    mask = h >= kth
    gnum = jnp.where(mask, e_all, 0.0)
    g_ref[...] = gnum / jnp.sum(gnum, axis=1, keepdims=True)
    act_ref[...] = jnp.max(mask.astype(jnp.float32), axis=0, keepdims=True)


def _sc_compact_body(act_hbm, idx_hbm, act_v, idx_v):
    cid = lax.axis_index("c")
    sid = lax.axis_index("s")

    @pl.when((cid == 0) & (sid == 0))
    def _compact():
        pltpu.sync_copy(act_hbm, act_v)
        count = jnp.int32(0)
        last = jnp.float32(-1.0)
        pos = []
        act = []
        for k in range(E // _LANE):
            a = act_v[0, pl.ds(k * _LANE, _LANE)]
            ai = jnp.where(a > 0.0, 1, 0).astype(jnp.int32)
            c = plsc.cumsum(ai) + count
            pos.append(c - 1)
            act.append(a > 0.0)
            count = count + jnp.sum(ai)
            iota_f = lax.iota(jnp.int32, _LANE).astype(jnp.float32)
            iota_f = iota_f + jnp.float32(k * _LANE)
            last = jnp.maximum(
                last, jnp.max(jnp.where(a > 0.0, iota_f, -1.0)))
        last_i = last.astype(jnp.int32)
        for k in range(E // _LANE):
            idx_v[pl.ds(k * _LANE, _LANE)] = (
                jnp.zeros((_LANE,), jnp.int32) + last_i)
        for k in range(E // _LANE):
            vals = lax.iota(jnp.int32, _LANE) + jnp.int32(k * _LANE)
            plsc.store_scatter(idx_v, [pos[k]], vals, mask=act[k])
        pltpu.sync_copy(idx_v, idx_hbm)


def _expert_body(idx_ref, x_ref, g_ref, w1_ref, b1_ref, w2_ref, b2_ref,
                 wf_ref, bf_ref, out_ref, acc_ref, xb_ref):
    i = pl.program_id(0)
    e = idx_ref[i]
    prev = idx_ref[jnp.maximum(i - 1, 0)]
    is_new = (i == 0) | (e != prev)

    @pl.when(i == 0)
    def _init():
        acc_ref[...] = jnp.zeros_like(acc_ref)
        xb_ref[...] = x_ref[...].astype(jnp.bfloat16)

    @pl.when(is_new)
    def _compute():
        w1 = w1_ref[0].astype(jnp.bfloat16)
        h1 = jnp.dot(xb_ref[...], w1, preferred_element_type=jnp.float32)
        h1 = jnp.maximum(h1 + b1_ref[0, 0], 0.0)
        w2 = w2_ref[0].astype(jnp.bfloat16)
        eo = jnp.dot(h1.astype(jnp.bfloat16), w2,
                     preferred_element_type=jnp.float32) + b2_ref[0, 0]
        lane = lax.broadcasted_iota(jnp.int32, (B, E), 1)
        gate = jnp.sum(jnp.where(lane == e, g_ref[...], 0.0), axis=1,
                       keepdims=True)
        acc_ref[...] += gate * eo

    @pl.when(i == E - 1)
    def _final():
        out_ref[...] = jnp.dot(acc_ref[...], wf_ref[...],
                               preferred_element_type=jnp.float32) + bf_ref[...]


def kernel(x_list, Wg, bg, W1, b1, W2, b2, Wf, bf):
    x = x_list.reshape(B, D)

    probs, G, act = pl.pallas_call(
        _gate_body,
        out_shape=(
            jax.ShapeDtypeStruct((B, E), jnp.float32),
            jax.ShapeDtypeStruct((B, E), jnp.float32),
            jax.ShapeDtypeStruct((1, E), jnp.float32),
        ),
    )(x, Wg, bg.reshape(1, E))

    sc_compact = pl.kernel(
        _sc_compact_body,
        out_type=jax.ShapeDtypeStruct((E,), jnp.int32),
        mesh=plsc.VectorSubcoreMesh(core_axis_name="c", subcore_axis_name="s"),
        compiler_params=pltpu.CompilerParams(needs_layout_passes=False),
        scratch_types=[
            pltpu.VMEM((1, E), jnp.float32),
            pltpu.VMEM((E,), jnp.int32),
        ],
    )
    idx = sc_compact(act)

    grid_spec = pltpu.PrefetchScalarGridSpec(
        num_scalar_prefetch=1,
        grid=(E,),
        in_specs=[
            pl.BlockSpec((B, D), lambda i, idx_ref: (0, 0)),
            pl.BlockSpec((B, E), lambda i, idx_ref: (0, 0)),
            pl.BlockSpec((1, D, HID), lambda i, idx_ref: (idx_ref[i], 0, 0)),
            pl.BlockSpec((1, 1, HID), lambda i, idx_ref: (idx_ref[i], 0, 0)),
            pl.BlockSpec((1, HID, HID), lambda i, idx_ref: (idx_ref[i], 0, 0)),
            pl.BlockSpec((1, 1, HID), lambda i, idx_ref: (idx_ref[i], 0, 0)),
            pl.BlockSpec((HID, OUT), lambda i, idx_ref: (0, 0)),
            pl.BlockSpec((1, OUT), lambda i, idx_ref: (0, 0)),
        ],
        out_specs=pl.BlockSpec((B, OUT), lambda i, idx_ref: (0, 0)),
        scratch_shapes=[
            pltpu.VMEM((B, HID), jnp.float32),
            pltpu.VMEM((B, D), jnp.bfloat16),
        ],
    )
    out = pl.pallas_call(
        _expert_body,
        grid_spec=grid_spec,
        out_shape=jax.ShapeDtypeStruct((B, OUT), jnp.float32),
    )(idx, x, G, W1, b1.reshape(E, 1, HID), W2, b2.reshape(E, 1, HID),
      Wf, bf.reshape(1, OUT))

    return (out, probs.reshape(1, B, E))
